# Initial kernel scaffold; baseline (speedup 1.0000x reference)
import functools

import jax
import jax.numpy as jnp
from jax import lax
from jax.experimental import pallas as pl
from jax.experimental.pallas import tpu as pltpu
from jax.experimental.pallas import tpu_sc as plsc

N = 10000
E = 320000
D = 128
L = 16
NC = 2
NS = 16
NW = NC * NS
NP = 10240
RPT = NP // NS
EPW = E // NW
NR = E // (NS * EPW)
C = 80
NCH = EPW // C
NEG_SLOPE = 0.2


def _proj_first_body(x_ref, w_ref, as_ref, ad_ref, h_ref, es_ref, ed_ref):
    h = jnp.dot(x_ref[...], w_ref[...], preferred_element_type=jnp.float32)
    h_ref[...] = h
    es_ref[...] = jnp.dot(h, as_ref[...], preferred_element_type=jnp.float32)
    ed_ref[...] = jnp.dot(h, ad_ref[...], preferred_element_type=jnp.float32)


def _proj_next_body(p_ref, b_ref, w_ref, as_ref, ad_ref, h_ref, es_ref, ed_ref):
    z = jnp.maximum(p_ref[0] + p_ref[1] + b_ref[...], 0.0)
    h = jnp.dot(z, w_ref[...], preferred_element_type=jnp.float32)
    h_ref[...] = h
    es_ref[...] = jnp.dot(h, as_ref[...], preferred_element_type=jnp.float32)
    ed_ref[...] = jnp.dot(h, ad_ref[...], preferred_element_type=jnp.float32)


def _final_body(p_ref, b_ref, o_ref):
    o_ref[...] = p_ref[0] + p_ref[1] + b_ref[...]


_PROJ_OUT = (
    jax.ShapeDtypeStruct((N, D), jnp.float32),
    jax.ShapeDtypeStruct((N, 1), jnp.float32),
    jax.ShapeDtypeStruct((N, 1), jnp.float32),
)


def _proj_first(x, W, a_s, a_d):
    return pl.pallas_call(_proj_first_body, out_shape=_PROJ_OUT)(
        x, W, a_s.reshape(D, 1), a_d.reshape(D, 1))


def _proj_next(p, b, W, a_s, a_d):
    return pl.pallas_call(_proj_next_body, out_shape=_PROJ_OUT)(
        p, b.reshape(1, D), W, a_s.reshape(D, 1), a_d.reshape(D, 1))


def _final(p, b):
    return pl.pallas_call(
        _final_body, out_shape=jax.ShapeDtypeStruct((N, D), jnp.float32))(
            p, b.reshape(1, D))


def _sc_edge_body(h_hbm, es_hbm, ed_hbm, src_hbm, dst_hbm, dst3_hbm, out_hbm,
                  es_v, ed_v, den_v, src_v, dst_v, dst2_v, alpha_v, rows_v,
                  den_sh, out_sh, sem):
    c = lax.axis_index("c")
    s = lax.axis_index("s")
    wid = s * NC + c
    zero16 = jnp.zeros((L,), jnp.float32)

    def _zden(i, carry):
        den_v[pl.ds(i * L, L)] = zero16
        return carry
    lax.fori_loop(0, NP // L, _zden, 0)

    def _zrow(r, carry):
        for k in range(D // L):
            rows_v[r, pl.ds(k * L, L)] = zero16
        return carry
    lax.fori_loop(0, C, _zrow, 0)

    pltpu.sync_copy(den_v.at[pl.ds(0, RPT)], den_sh.at[pl.ds(s * RPT, RPT)])
    for q in range(RPT // C):
        pltpu.sync_copy(rows_v, out_sh.at[pl.ds(s * RPT + q * C, C)])

    pltpu.sync_copy(es_hbm, es_v)
    pltpu.sync_copy(ed_hbm, ed_v)
    plsc.subcore_barrier()

    for r in range(NR):
        base_a = (s * NR + r) * EPW
        pltpu.sync_copy(src_hbm.at[pl.ds(base_a, EPW)], src_v)
        pltpu.sync_copy(dst_hbm.at[pl.ds(base_a, EPW)], dst_v)

        def _step_a(i, carry):
            isrc = src_v[pl.ds(i * L, L)]
            idst = dst_v[pl.ds(i * L, L)]
            t = plsc.load_gather(es_v, [isrc]) + plsc.load_gather(ed_v, [idst])
            lg = jnp.where(t > 0.0, t, t * NEG_SLOPE)
            plsc.addupdate_scatter(den_v, [idst], jnp.exp(lg))
            return carry
        lax.fori_loop(0, EPW // L, _step_a, 0)

    pltpu.sync_copy(den_v, den_sh, add=True)
    plsc.subcore_barrier()
    pltpu.sync_copy(den_sh, den_v)

    base_b = wid * EPW
    pltpu.sync_copy(src_hbm.at[pl.ds(base_b, EPW)], src_v)
    pltpu.sync_copy(dst_hbm.at[pl.ds(base_b, EPW)], dst_v)
    pltpu.sync_copy(dst3_hbm.at[wid], dst2_v)

    def _step_b(i, carry):
        isrc = src_v[pl.ds(i * L, L)]
        idst = dst_v[pl.ds(i * L, L)]
        t = plsc.load_gather(es_v, [isrc]) + plsc.load_gather(ed_v, [idst])
        lg = jnp.where(t > 0.0, t, t * NEG_SLOPE)
        ex = jnp.exp(lg)
        dg = plsc.load_gather(den_v, [idst])
        alpha_v[pl.ds(i * L, L)] = ex / (dg + 1e-16)
        return carry
    lax.fori_loop(0, EPW // L, _step_b, 0)

    def _chunk(j, carry):
        pltpu.async_copy(h_hbm.at[src_v.at[pl.ds(j * C, C)]], rows_v, sem).wait()

        def _scale(r, carry2):
            a = plsc.load_gather(alpha_v, [jnp.full((L,), j * C + r, jnp.int32)])
            for k in range(D // L):
                rows_v[r, pl.ds(k * L, L)] = rows_v[r, pl.ds(k * L, L)] * a
            return carry2
        lax.fori_loop(0, C, _scale, 0)

        pltpu.sync_copy(rows_v, out_sh.at[dst2_v.at[j]], add=True)
        return carry
    lax.fori_loop(0, NCH, _chunk, 0)

    plsc.subcore_barrier()
    pltpu.sync_copy(out_sh.at[pl.ds(s * RPT, RPT)],
                    out_hbm.at[c, pl.ds(s * RPT, RPT)])


_sc_edge = pl.kernel(
    _sc_edge_body,
    out_type=jax.ShapeDtypeStruct((NC, NP, D), jnp.float32),
    mesh=plsc.VectorSubcoreMesh(core_axis_name="c", subcore_axis_name="s"),
    scratch_types=[
        pltpu.VMEM((N,), jnp.float32),
        pltpu.VMEM((N,), jnp.float32),
        pltpu.VMEM((NP,), jnp.float32),
        pltpu.VMEM((EPW,), jnp.int32),
        pltpu.VMEM((EPW,), jnp.int32),
        pltpu.VMEM((NCH, C), jnp.int32),
        pltpu.VMEM((EPW,), jnp.float32),
        pltpu.VMEM((C, D), jnp.float32),
        pltpu.VMEM_SHARED((NP,), jnp.float32),
        pltpu.VMEM_SHARED((NP, D), jnp.float32),
        pltpu.SemaphoreType.DMA,
    ],
)


def kernel(x, edge_index, W1, a_src1, a_dst1, b1, W2, a_src2, a_dst2, b2,
           W3, a_src3, a_dst3, b3):
    src = edge_index[0]
    dst = edge_index[1]
    dst3 = dst.reshape(NW, NCH, C)

    h, es, ed = _proj_first(x, W1, a_src1, a_dst1)
    p = _sc_edge(h, es.reshape(N), ed.reshape(N), src, dst, dst3)[:, :N, :]

    h, es, ed = _proj_next(p, b1, W2, a_src2, a_dst2)
    p = _sc_edge(h, es.reshape(N), ed.reshape(N), src, dst, dst3)[:, :N, :]

    h, es, ed = _proj_next(p, b2, W3, a_src3, a_dst3)
    p = _sc_edge(h, es.reshape(N), ed.reshape(N), src, dst, dst3)[:, :N, :]

    return _final(p, b3)


# speedup vs baseline: 23.2390x; 23.2390x over previous
import functools

import jax
import jax.numpy as jnp
from jax import lax
from jax.experimental import pallas as pl
from jax.experimental.pallas import tpu as pltpu
from jax.experimental.pallas import tpu_sc as plsc

N = 10000
E = 320000
D = 128
L = 16
NC = 2
NS = 16
NW = NC * NS
NP = 10112
RPT = NP // NS
EPW = E // NW
NR = E // (NS * EPW)
C = 80
EB = 2000
DEN_R = 80
DEN_C = 128
NEG_SLOPE = 0.2


def _proj_first_body(x_ref, w_ref, as_ref, ad_ref, h_ref, es_ref, ed_ref):
    h = jnp.dot(x_ref[...], w_ref[...], preferred_element_type=jnp.float32)
    h_ref[...] = h
    es_ref[...] = jnp.dot(h, as_ref[...], preferred_element_type=jnp.float32)
    ed_ref[...] = jnp.dot(h, ad_ref[...], preferred_element_type=jnp.float32)


def _proj_next_body(p_ref, b_ref, w_ref, as_ref, ad_ref, h_ref, es_ref, ed_ref):
    z = jnp.maximum(p_ref[0] + p_ref[1] + b_ref[...], 0.0)
    h = jnp.dot(z, w_ref[...], preferred_element_type=jnp.float32)
    h_ref[...] = h
    es_ref[...] = jnp.dot(h, as_ref[...], preferred_element_type=jnp.float32)
    ed_ref[...] = jnp.dot(h, ad_ref[...], preferred_element_type=jnp.float32)


def _final_body(p_ref, b_ref, o_ref):
    o_ref[...] = p_ref[0] + p_ref[1] + b_ref[...]


_PROJ_OUT = (
    jax.ShapeDtypeStruct((N, D), jnp.float32),
    jax.ShapeDtypeStruct((N, 1), jnp.float32),
    jax.ShapeDtypeStruct((N, 1), jnp.float32),
)


def _proj_first(x, W, a_s, a_d):
    return pl.pallas_call(_proj_first_body, out_shape=_PROJ_OUT)(
        x, W, a_s.reshape(D, 1), a_d.reshape(D, 1))


def _proj_next(p, b, W, a_s, a_d):
    return pl.pallas_call(_proj_next_body, out_shape=_PROJ_OUT)(
        p, b.reshape(1, D), W, a_s.reshape(D, 1), a_d.reshape(D, 1))


def _final(p, b):
    return pl.pallas_call(
        _final_body, out_shape=jax.ShapeDtypeStruct((N, D), jnp.float32))(
            p, b.reshape(1, D))


def _att(es_v, ed_v, isrc, idst):
    t = plsc.load_gather(es_v, [isrc]) + plsc.load_gather(ed_v, [idst])
    lg = jnp.where(t > 0.0, t, t * NEG_SLOPE)
    return jnp.exp(lg)


def _den_idx(idst):
    return [lax.shift_right_logical(idst, 7), lax.bitwise_and(idst, 127)]


def _sc_edge_body(h_hbm, es_hbm, ed_hbm, src_hbm, dst_hbm, out_hbm,
                  es_v, ed_v, den_v, src_v, dst_v, alpha_c, rows_v,
                  den_sh, out_sh, sem):
    c = lax.axis_index("c")
    s = lax.axis_index("s")
    wid = s * NC + c
    zero16 = jnp.zeros((L,), jnp.float32)

    def _zden(r, carry):
        for k in range(DEN_C // L):
            den_v[r, pl.ds(k * L, L)] = zero16
        return carry
    lax.fori_loop(0, DEN_R, _zden, 0)

    def _zrow(r, carry):
        for k in range(D // L):
            rows_v[r, pl.ds(k * L, L)] = zero16
        return carry
    lax.fori_loop(0, C, _zrow, 0)

    pltpu.sync_copy(den_v.at[pl.ds(s * (DEN_R // NS), DEN_R // NS)],
                    den_sh.at[pl.ds(s * (DEN_R // NS), DEN_R // NS)])
    for q in range(RPT // C):
        pltpu.sync_copy(rows_v, out_sh.at[pl.ds(s * RPT + q * C, C)])
    if RPT % C:
        pltpu.sync_copy(rows_v.at[pl.ds(0, RPT % C)],
                        out_sh.at[pl.ds(s * RPT + (RPT // C) * C, RPT % C)])

    pltpu.sync_copy(es_hbm, es_v)
    pltpu.sync_copy(ed_hbm, ed_v)
    plsc.subcore_barrier()

    for t in range(E // (NS * EB)):
        base_a = s * (E // NS) + t * EB
        pltpu.sync_copy(src_hbm.at[pl.ds(base_a, EB)], src_v)
        pltpu.sync_copy(dst_hbm.at[pl.ds(base_a, EB)], dst_v)

        def _step_a(i, carry):
            isrc = src_v[pl.ds(i * L, L)]
            idst = dst_v[pl.ds(i * L, L)]
            plsc.addupdate_scatter(den_v, _den_idx(idst),
                                   _att(es_v, ed_v, isrc, idst))
            return carry
        lax.fori_loop(0, EB // L, _step_a, 0)

    for k in range(DEN_R // L):
        rows16 = lax.iota(jnp.int32, L) + k * L
        pltpu.sync_copy(den_v.at[pl.ds(k * L, L)],
                        den_sh.at[rows16], add=True)
    plsc.subcore_barrier()
    pltpu.sync_copy(den_sh, den_v)

    for t in range(EPW // EB):
        base_b = wid * EPW + t * EB
        pltpu.sync_copy(src_hbm.at[pl.ds(base_b, EB)], src_v)
        pltpu.sync_copy(dst_hbm.at[pl.ds(base_b, EB)], dst_v)

        def _chunk(j, carry):
            pltpu.async_copy(h_hbm.at[src_v.at[pl.ds(j * C, C)]],
                             rows_v, sem).wait()
            for k in range(C // L):
                isrc = src_v[pl.ds(j * C + k * L, L)]
                idst = dst_v[pl.ds(j * C + k * L, L)]
                ex = _att(es_v, ed_v, isrc, idst)
                dg = plsc.load_gather(den_v, _den_idx(idst))
                alpha_c[pl.ds(k * L, L)] = ex / (dg + 1e-16)

            def _scale(r, carry2):
                a = plsc.load_gather(alpha_c, [jnp.full((L,), r, jnp.int32)])
                for k in range(D // L):
                    rows_v[r, pl.ds(k * L, L)] = rows_v[r, pl.ds(k * L, L)] * a
                return carry2
            lax.fori_loop(0, C, _scale, 0)

            for k in range(C // L):
                idst = dst_v[pl.ds(j * C + k * L, L)]
                pltpu.sync_copy(rows_v.at[pl.ds(k * L, L)],
                                out_sh.at[idst], add=True)
            return carry
        lax.fori_loop(0, EB // C, _chunk, 0)

    plsc.subcore_barrier()
    pltpu.sync_copy(out_sh.at[pl.ds(s * RPT, RPT)],
                    out_hbm.at[c, pl.ds(s * RPT, RPT)])


_sc_edge = pl.kernel(
    _sc_edge_body,
    out_type=jax.ShapeDtypeStruct((NC, NP, D), jnp.float32),
    mesh=plsc.VectorSubcoreMesh(core_axis_name="c", subcore_axis_name="s"),
    scratch_types=[
        pltpu.VMEM((N,), jnp.float32),
        pltpu.VMEM((N,), jnp.float32),
        pltpu.VMEM((DEN_R, DEN_C), jnp.float32),
        pltpu.VMEM((EB,), jnp.int32),
        pltpu.VMEM((EB,), jnp.int32),
        pltpu.VMEM((C,), jnp.float32),
        pltpu.VMEM((C, D), jnp.float32),
        pltpu.VMEM_SHARED((DEN_R, DEN_C), jnp.float32),
        pltpu.VMEM_SHARED((NP, D), jnp.float32),
        pltpu.SemaphoreType.DMA,
    ],
    compiler_params=pltpu.CompilerParams(needs_layout_passes=False),
)


def kernel(x, edge_index, W1, a_src1, a_dst1, b1, W2, a_src2, a_dst2, b2,
           W3, a_src3, a_dst3, b3):
    src = edge_index[0]
    dst = edge_index[1]

    h, es, ed = _proj_first(x, W1, a_src1, a_dst1)
    p = _sc_edge(h, es.reshape(N), ed.reshape(N), src, dst)[:, :N, :]

    h, es, ed = _proj_next(p, b1, W2, a_src2, a_dst2)
    p = _sc_edge(h, es.reshape(N), ed.reshape(N), src, dst)[:, :N, :]

    h, es, ed = _proj_next(p, b2, W3, a_src3, a_dst3)
    p = _sc_edge(h, es.reshape(N), ed.reshape(N), src, dst)[:, :N, :]

    return _final(p, b3)


# speedup vs baseline: 24.0021x; 1.0328x over previous
import functools

import jax
import jax.numpy as jnp
from jax import lax
from jax.experimental import pallas as pl
from jax.experimental.pallas import tpu as pltpu
from jax.experimental.pallas import tpu_sc as plsc

N = 10000
E = 320000
D = 128
L = 16
NC = 2
NS = 16
NW = NC * NS
NP = 10112
RPT = NP // NS
EPT = E // NS
EPW = E // NW
C = 80
EB = 2000
NCH = EB // C
DEN_R = 80
DEN_C = 128
NEG_SLOPE = 0.2


def _proj_first_body(x_ref, w_ref, as_ref, ad_ref, h_ref, es_ref, ed_ref):
    h = jnp.dot(x_ref[...], w_ref[...], preferred_element_type=jnp.float32)
    h_ref[...] = h
    es_ref[...] = jnp.dot(h, as_ref[...], preferred_element_type=jnp.float32)
    ed_ref[...] = jnp.dot(h, ad_ref[...], preferred_element_type=jnp.float32)


def _proj_next_body(p_ref, b_ref, w_ref, as_ref, ad_ref, h_ref, es_ref, ed_ref):
    z = jnp.maximum(p_ref[0] + p_ref[1] + b_ref[...], 0.0)
    h = jnp.dot(z, w_ref[...], preferred_element_type=jnp.float32)
    h_ref[...] = h
    es_ref[...] = jnp.dot(h, as_ref[...], preferred_element_type=jnp.float32)
    ed_ref[...] = jnp.dot(h, ad_ref[...], preferred_element_type=jnp.float32)


def _final_body(p_ref, b_ref, o_ref):
    o_ref[...] = p_ref[0] + p_ref[1] + b_ref[...]


_PROJ_OUT = (
    jax.ShapeDtypeStruct((N, D), jnp.float32),
    jax.ShapeDtypeStruct((N, 1), jnp.float32),
    jax.ShapeDtypeStruct((N, 1), jnp.float32),
)


def _proj_first(x, W, a_s, a_d):
    return pl.pallas_call(_proj_first_body, out_shape=_PROJ_OUT)(
        x, W, a_s.reshape(D, 1), a_d.reshape(D, 1))


def _proj_next(p, b, W, a_s, a_d):
    return pl.pallas_call(_proj_next_body, out_shape=_PROJ_OUT)(
        p, b.reshape(1, D), W, a_s.reshape(D, 1), a_d.reshape(D, 1))


def _final(p, b):
    return pl.pallas_call(
        _final_body, out_shape=jax.ShapeDtypeStruct((N, D), jnp.float32))(
            p, b.reshape(1, D))


def _den_idx(idst):
    return [lax.shift_right_logical(idst, 7), lax.bitwise_and(idst, 127)]


def _sc_edge_body(h_hbm, es_hbm, ed_hbm, src_hbm, dst_hbm, out_hbm,
                  den_v, src_v, dst_v, esg_v, edg_v, alpha_c,
                  rows_a, rows_b, den_sh, out_sh, sem_a, sem_b, sem_g):
    c = lax.axis_index("c")
    s = lax.axis_index("s")
    wid = s * NC + c
    zero16 = jnp.zeros((L,), jnp.float32)

    def _zden(r, carry):
        for k in range(DEN_C // L):
            den_v[r, pl.ds(k * L, L)] = zero16
        return carry
    lax.fori_loop(0, DEN_R, _zden, 0)

    def _zrow(r, carry):
        for k in range(D // L):
            rows_a[r, pl.ds(k * L, L)] = zero16
        return carry
    lax.fori_loop(0, C, _zrow, 0)

    pltpu.sync_copy(den_v.at[pl.ds(s * (DEN_R // NS), DEN_R // NS)],
                    den_sh.at[pl.ds(s * (DEN_R // NS), DEN_R // NS)])
    for q in range(RPT // C):
        pltpu.sync_copy(rows_a, out_sh.at[pl.ds(s * RPT + q * C, C)])
    if RPT % C:
        pltpu.sync_copy(rows_a.at[pl.ds(0, RPT % C)],
                        out_sh.at[pl.ds(s * RPT + (RPT // C) * C, RPT % C)])

    def _stage_piece(base):
        pltpu.sync_copy(src_hbm.at[pl.ds(base, EB)], src_v)
        pltpu.sync_copy(dst_hbm.at[pl.ds(base, EB)], dst_v)
        pltpu.async_copy(es_hbm.at[src_v], esg_v, sem_g)
        pltpu.async_copy(ed_hbm.at[dst_v], edg_v, sem_g)
        pltpu.make_async_copy(es_hbm.at[pl.ds(0, EB)], esg_v, sem_g).wait()
        pltpu.make_async_copy(ed_hbm.at[pl.ds(0, EB)], edg_v, sem_g).wait()

    def _ex_vec(i):
        t = esg_v[pl.ds(i * L, L)] + edg_v[pl.ds(i * L, L)]
        lg = jnp.where(t > 0.0, t, t * NEG_SLOPE)
        return jnp.exp(lg)

    for t in range(EPT // EB):
        _stage_piece(s * EPT + t * EB)

        def _step_a(i, carry):
            idst = dst_v[pl.ds(i * L, L)]
            plsc.addupdate_scatter(den_v, _den_idx(idst), _ex_vec(i))
            return carry
        lax.fori_loop(0, EB // L, _step_a, 0)

    for k in range(DEN_R // L):
        rows16 = lax.iota(jnp.int32, L) + k * L
        pltpu.sync_copy(den_v.at[pl.ds(k * L, L)],
                        den_sh.at[rows16], add=True)
    plsc.subcore_barrier()
    pltpu.sync_copy(den_sh, den_v)

    def _gather(j, buf, sem):
        pltpu.async_copy(h_hbm.at[src_v.at[pl.ds(j * C, C)]], buf, sem)

    def _drain(buf, sem):
        pltpu.make_async_copy(h_hbm.at[pl.ds(0, C)], buf, sem).wait()

    def _proc(j, buf):
        for k in range(C // L):
            idst = dst_v[pl.ds(j * C + k * L, L)]
            ex = _ex_vec(j * (C // L) + k)
            dg = plsc.load_gather(den_v, _den_idx(idst))
            alpha_c[pl.ds(k * L, L)] = ex / (dg + 1e-16)

        def _scale(r, carry2):
            a = plsc.load_gather(alpha_c, [jnp.full((L,), r, jnp.int32)])
            for k in range(D // L):
                buf[r, pl.ds(k * L, L)] = buf[r, pl.ds(k * L, L)] * a
            return carry2
        lax.fori_loop(0, C, _scale, 0)

        for k in range(C // L):
            idst = dst_v[pl.ds(j * C + k * L, L)]
            pltpu.sync_copy(buf.at[pl.ds(k * L, L)],
                            out_sh.at[idst], add=True)

    for t in range(EPW // EB):
        _stage_piece(wid * EPW + t * EB)

        _gather(0, rows_a, sem_a)

        def _pair(i, carry):
            j0 = 2 * i
            _drain(rows_a, sem_a)
            _gather(j0 + 1, rows_b, sem_b)
            _proc(j0, rows_a)
            _drain(rows_b, sem_b)
            _gather(j0 + 2, rows_a, sem_a)
            _proc(j0 + 1, rows_b)
            return carry
        lax.fori_loop(0, (NCH - 1) // 2, _pair, 0)
        _drain(rows_a, sem_a)
        _proc(NCH - 1, rows_a)

    plsc.subcore_barrier()
    pltpu.sync_copy(out_sh.at[pl.ds(s * RPT, RPT)],
                    out_hbm.at[c, pl.ds(s * RPT, RPT)])


_sc_edge = pl.kernel(
    _sc_edge_body,
    out_type=jax.ShapeDtypeStruct((NC, NP, D), jnp.float32),
    mesh=plsc.VectorSubcoreMesh(core_axis_name="c", subcore_axis_name="s"),
    scratch_types=[
        pltpu.VMEM((DEN_R, DEN_C), jnp.float32),
        pltpu.VMEM((EB,), jnp.int32),
        pltpu.VMEM((EB,), jnp.int32),
        pltpu.VMEM((EB,), jnp.float32),
        pltpu.VMEM((EB,), jnp.float32),
        pltpu.VMEM((C,), jnp.float32),
        pltpu.VMEM((C, D), jnp.float32),
        pltpu.VMEM((C, D), jnp.float32),
        pltpu.VMEM_SHARED((DEN_R, DEN_C), jnp.float32),
        pltpu.VMEM_SHARED((NP, D), jnp.float32),
        pltpu.SemaphoreType.DMA,
        pltpu.SemaphoreType.DMA,
        pltpu.SemaphoreType.DMA,
    ],
    compiler_params=pltpu.CompilerParams(needs_layout_passes=False),
)


def kernel(x, edge_index, W1, a_src1, a_dst1, b1, W2, a_src2, a_dst2, b2,
           W3, a_src3, a_dst3, b3):
    src = edge_index[0]
    dst = edge_index[1]

    h, es, ed = _proj_first(x, W1, a_src1, a_dst1)
    p = _sc_edge(h, es.reshape(N), ed.reshape(N), src, dst)[:, :N, :]

    h, es, ed = _proj_next(p, b1, W2, a_src2, a_dst2)
    p = _sc_edge(h, es.reshape(N), ed.reshape(N), src, dst)[:, :N, :]

    h, es, ed = _proj_next(p, b2, W3, a_src3, a_dst3)
    p = _sc_edge(h, es.reshape(N), ed.reshape(N), src, dst)[:, :N, :]

    return _final(p, b3)


# speedup vs baseline: 27.9841x; 1.1659x over previous
import functools

import jax
import jax.numpy as jnp
from jax import lax
from jax.experimental import pallas as pl
from jax.experimental.pallas import tpu as pltpu
from jax.experimental.pallas import tpu_sc as plsc

N = 10000
E = 320000
D = 128
L = 16
NC = 2
NS = 16
NW = NC * NS
NP = 10112
RPT = NP // NS
EPT = E // NS
EPW = E // NW
C = 80
EB = 2000
NCH = EB // C
DEN_R = 80
DEN_C = 128
NEG_SLOPE = 0.2


def _proj_first_body(x_ref, w_ref, as_ref, ad_ref, h_ref, es_ref, ed_ref):
    h = jnp.dot(x_ref[...], w_ref[...], preferred_element_type=jnp.float32)
    h_ref[...] = h
    es_ref[...] = jnp.dot(h, as_ref[...], preferred_element_type=jnp.float32)
    ed_ref[...] = jnp.dot(h, ad_ref[...], preferred_element_type=jnp.float32)


def _proj_next_body(p_ref, b_ref, w_ref, as_ref, ad_ref, h_ref, es_ref, ed_ref):
    z = jnp.maximum(p_ref[0] + p_ref[1] + b_ref[...], 0.0)
    h = jnp.dot(z, w_ref[...], preferred_element_type=jnp.float32)
    h_ref[...] = h
    es_ref[...] = jnp.dot(h, as_ref[...], preferred_element_type=jnp.float32)
    ed_ref[...] = jnp.dot(h, ad_ref[...], preferred_element_type=jnp.float32)


def _final_body(p_ref, b_ref, o_ref):
    o_ref[...] = p_ref[0] + p_ref[1] + b_ref[...]


_PROJ_OUT = (
    jax.ShapeDtypeStruct((N, D), jnp.float32),
    jax.ShapeDtypeStruct((N, 1), jnp.float32),
    jax.ShapeDtypeStruct((N, 1), jnp.float32),
)


def _proj_first(x, W, a_s, a_d):
    return pl.pallas_call(_proj_first_body, out_shape=_PROJ_OUT)(
        x, W, a_s.reshape(D, 1), a_d.reshape(D, 1))


def _proj_next(p, b, W, a_s, a_d):
    return pl.pallas_call(_proj_next_body, out_shape=_PROJ_OUT)(
        p, b.reshape(1, D), W, a_s.reshape(D, 1), a_d.reshape(D, 1))


def _final(p, b):
    return pl.pallas_call(
        _final_body, out_shape=jax.ShapeDtypeStruct((N, D), jnp.float32))(
            p, b.reshape(1, D))


def _den_idx(idst):
    return [lax.shift_right_logical(idst, 7), lax.bitwise_and(idst, 127)]


def _sc_edge_body(h_hbm, es_hbm, ed_hbm, src_hbm, dst_hbm, out_hbm,
                  den_v, src_v, dst_v, esg_v, edg_v, alpha_c,
                  rows_a, rows_b, rows_c, den_sh, out_sh,
                  sem_g, sem_s0, sem_s1, sem_s2):
    c = lax.axis_index("c")
    s = lax.axis_index("s")
    wid = s * NC + c
    zero16 = jnp.zeros((L,), jnp.float32)

    def _zden(r, carry):
        for k in range(DEN_C // L):
            den_v[r, pl.ds(k * L, L)] = zero16
        return carry
    lax.fori_loop(0, DEN_R, _zden, 0)

    def _zrow(r, carry):
        for k in range(D // L):
            rows_a[r, pl.ds(k * L, L)] = zero16
        return carry
    lax.fori_loop(0, C, _zrow, 0)

    pltpu.sync_copy(den_v.at[pl.ds(s * (DEN_R // NS), DEN_R // NS)],
                    den_sh.at[pl.ds(s * (DEN_R // NS), DEN_R // NS)])
    for q in range(RPT // C):
        pltpu.sync_copy(rows_a, out_sh.at[pl.ds(s * RPT + q * C, C)])
    if RPT % C:
        pltpu.sync_copy(rows_a.at[pl.ds(0, RPT % C)],
                        out_sh.at[pl.ds(s * RPT + (RPT // C) * C, RPT % C)])

    def _stage_piece(base):
        pltpu.sync_copy(src_hbm.at[pl.ds(base, EB)], src_v)
        pltpu.sync_copy(dst_hbm.at[pl.ds(base, EB)], dst_v)
        pltpu.async_copy(es_hbm.at[src_v], esg_v, sem_g)
        pltpu.async_copy(ed_hbm.at[dst_v], edg_v, sem_g)
        pltpu.make_async_copy(es_hbm.at[pl.ds(0, EB)], esg_v, sem_g).wait()
        pltpu.make_async_copy(ed_hbm.at[pl.ds(0, EB)], edg_v, sem_g).wait()

    def _ex_vec(i):
        t = esg_v[pl.ds(i * L, L)] + edg_v[pl.ds(i * L, L)]
        lg = jnp.where(t > 0.0, t, t * NEG_SLOPE)
        return jnp.exp(lg)

    for t in range(EPT // EB):
        _stage_piece(s * EPT + t * EB)

        def _step_a(i, carry):
            idst = dst_v[pl.ds(i * L, L)]
            plsc.addupdate_scatter(den_v, _den_idx(idst), _ex_vec(i))
            return carry
        lax.fori_loop(0, EB // L, _step_a, 0)

    for k in range(DEN_R // L):
        rows16 = lax.iota(jnp.int32, L) + k * L
        pltpu.sync_copy(den_v.at[pl.ds(k * L, L)],
                        den_sh.at[rows16], add=True)
    plsc.subcore_barrier()
    pltpu.sync_copy(den_sh, den_v)

    def _gather(j, buf):
        pltpu.async_copy(h_hbm.at[src_v.at[pl.ds(j * C, C)]], buf, sem_g)

    def _drain(buf, sem):
        pltpu.make_async_copy(h_hbm.at[pl.ds(0, C)], buf, sem).wait()

    def _scatter(j, buf, sem):
        pltpu.async_copy(buf, out_sh.at[dst_v.at[pl.ds(j * C, C)]],
                         sem, add=True)

    def _compute(j, buf):
        for k in range(C // L):
            idst = dst_v[pl.ds(j * C + k * L, L)]
            ex = _ex_vec(j * (C // L) + k)
            dg = plsc.load_gather(den_v, _den_idx(idst))
            alpha_c[pl.ds(k * L, L)] = ex / (dg + 1e-16)

        def _scale(r, carry2):
            a = plsc.load_gather(alpha_c, [jnp.full((L,), r, jnp.int32)])
            for k in range(D // L):
                buf[r, pl.ds(k * L, L)] = buf[r, pl.ds(k * L, L)] * a
            return carry2
        lax.fori_loop(0, C, _scale, 0)

    for t in range(EPW // EB):
        _stage_piece(wid * EPW + t * EB)

        _gather(0, rows_a)

        def _trip(i, carry):
            j0 = 3 * i

            @pl.when(j0 >= 2)
            def _():
                _drain(rows_b, sem_s1)

            _drain(rows_a, sem_g)
            _gather(j0 + 1, rows_b)
            _compute(j0, rows_a)
            _scatter(j0, rows_a, sem_s0)

            @pl.when(j0 + 1 >= 2)
            def _():
                _drain(rows_c, sem_s2)

            _drain(rows_b, sem_g)
            _gather(j0 + 2, rows_c)
            _compute(j0 + 1, rows_b)
            _scatter(j0 + 1, rows_b, sem_s1)

            _drain(rows_a, sem_s0)
            _drain(rows_c, sem_g)
            _gather(j0 + 3, rows_a)
            _compute(j0 + 2, rows_c)
            _scatter(j0 + 2, rows_c, sem_s2)
            return carry
        lax.fori_loop(0, (NCH - 1) // 3, _trip, 0)
        _drain(rows_b, sem_s1)
        _drain(rows_a, sem_g)
        _compute(NCH - 1, rows_a)
        _scatter(NCH - 1, rows_a, sem_s0)
        _drain(rows_c, sem_s2)
        _drain(rows_a, sem_s0)

    plsc.subcore_barrier()
    pltpu.sync_copy(out_sh.at[pl.ds(s * RPT, RPT)],
                    out_hbm.at[c, pl.ds(s * RPT, RPT)])


_sc_edge = pl.kernel(
    _sc_edge_body,
    out_type=jax.ShapeDtypeStruct((NC, NP, D), jnp.float32),
    mesh=plsc.VectorSubcoreMesh(core_axis_name="c", subcore_axis_name="s"),
    scratch_types=[
        pltpu.VMEM((DEN_R, DEN_C), jnp.float32),
        pltpu.VMEM((EB,), jnp.int32),
        pltpu.VMEM((EB,), jnp.int32),
        pltpu.VMEM((EB,), jnp.float32),
        pltpu.VMEM((EB,), jnp.float32),
        pltpu.VMEM((C,), jnp.float32),
        pltpu.VMEM((C, D), jnp.float32),
        pltpu.VMEM((C, D), jnp.float32),
        pltpu.VMEM((C, D), jnp.float32),
        pltpu.VMEM_SHARED((DEN_R, DEN_C), jnp.float32),
        pltpu.VMEM_SHARED((NP, D), jnp.float32),
        pltpu.SemaphoreType.DMA,
        pltpu.SemaphoreType.DMA,
        pltpu.SemaphoreType.DMA,
        pltpu.SemaphoreType.DMA,
    ],
    compiler_params=pltpu.CompilerParams(needs_layout_passes=False),
)


def kernel(x, edge_index, W1, a_src1, a_dst1, b1, W2, a_src2, a_dst2, b2,
           W3, a_src3, a_dst3, b3):
    src = edge_index[0]
    dst = edge_index[1]

    h, es, ed = _proj_first(x, W1, a_src1, a_dst1)
    p = _sc_edge(h, es.reshape(N), ed.reshape(N), src, dst)[:, :N, :]

    h, es, ed = _proj_next(p, b1, W2, a_src2, a_dst2)
    p = _sc_edge(h, es.reshape(N), ed.reshape(N), src, dst)[:, :N, :]

    h, es, ed = _proj_next(p, b2, W3, a_src3, a_dst3)
    p = _sc_edge(h, es.reshape(N), ed.reshape(N), src, dst)[:, :N, :]

    return _final(p, b3)


# speedup vs baseline: 36.3360x; 1.2985x over previous
import functools

import jax
import jax.numpy as jnp
from jax import lax
from jax.experimental import pallas as pl
from jax.experimental.pallas import tpu as pltpu
from jax.experimental.pallas import tpu_sc as plsc

N = 10000
E = 320000
D = 128
L = 16
NC = 2
NS = 16
NW = NC * NS
NP = 10112
RPT = NP // NS
EPW = E // NW
C = 80
EB = 2000
NCH = EB // C
DEN_R = 80
DEN_C = 128
NEG_SLOPE = 0.2
EPS = 1e-16


def _proj_first_body(x_ref, w_ref, as_ref, ad_ref, h_ref, es_ref, ed_ref):
    h = jnp.dot(x_ref[...], w_ref[...], preferred_element_type=jnp.float32)
    h_ref[...] = h
    es_ref[...] = jnp.dot(h, as_ref[...], preferred_element_type=jnp.float32)
    ed_ref[...] = jnp.dot(h, ad_ref[...], preferred_element_type=jnp.float32)


def _proj_next_body(p_ref, d_ref, b_ref, w_ref, as_ref, ad_ref,
                    h_ref, es_ref, ed_ref):
    den = d_ref[0] + d_ref[1] + EPS
    z = jnp.maximum((p_ref[0] + p_ref[1]) / den + b_ref[...], 0.0)
    h = jnp.dot(z, w_ref[...], preferred_element_type=jnp.float32)
    h_ref[...] = h
    es_ref[...] = jnp.dot(h, as_ref[...], preferred_element_type=jnp.float32)
    ed_ref[...] = jnp.dot(h, ad_ref[...], preferred_element_type=jnp.float32)


def _final_body(p_ref, d_ref, b_ref, o_ref):
    den = d_ref[0] + d_ref[1] + EPS
    o_ref[...] = (p_ref[0] + p_ref[1]) / den + b_ref[...]


_PROJ_OUT = (
    jax.ShapeDtypeStruct((N, D), jnp.float32),
    jax.ShapeDtypeStruct((N, 1), jnp.float32),
    jax.ShapeDtypeStruct((N, 1), jnp.float32),
)


def _proj_first(x, W, a_s, a_d):
    return pl.pallas_call(_proj_first_body, out_shape=_PROJ_OUT)(
        x, W, a_s.reshape(D, 1), a_d.reshape(D, 1))


def _proj_next(p, d, b, W, a_s, a_d):
    return pl.pallas_call(_proj_next_body, out_shape=_PROJ_OUT)(
        p, d, b.reshape(1, D), W, a_s.reshape(D, 1), a_d.reshape(D, 1))


def _final(p, d, b):
    return pl.pallas_call(
        _final_body, out_shape=jax.ShapeDtypeStruct((N, D), jnp.float32))(
            p, d, b.reshape(1, D))


def _den_idx(idst):
    return [lax.shift_right_logical(idst, 7), lax.bitwise_and(idst, 127)]


def _sc_edge_body(h_hbm, es_hbm, ed_hbm, src_hbm, dst_hbm,
                  out_hbm, dout_hbm,
                  den_v, src_v, dst_v, esg_v, edg_v, ex_c,
                  rows_a, rows_b, rows_c, den_sh, out_sh,
                  sem_g, sem_s0, sem_s1, sem_s2):
    c = lax.axis_index("c")
    s = lax.axis_index("s")
    wid = s * NC + c
    zero16 = jnp.zeros((L,), jnp.float32)

    def _zden(r, carry):
        for k in range(DEN_C // L):
            den_v[r, pl.ds(k * L, L)] = zero16
        return carry
    lax.fori_loop(0, DEN_R, _zden, 0)

    def _zrow(r, carry):
        for k in range(D // L):
            rows_a[r, pl.ds(k * L, L)] = zero16
        return carry
    lax.fori_loop(0, C, _zrow, 0)

    pltpu.sync_copy(den_v.at[pl.ds(s * (DEN_R // NS), DEN_R // NS)],
                    den_sh.at[pl.ds(s * (DEN_R // NS), DEN_R // NS)])
    for q in range(RPT // C):
        pltpu.sync_copy(rows_a, out_sh.at[pl.ds(s * RPT + q * C, C)])
    if RPT % C:
        pltpu.sync_copy(rows_a.at[pl.ds(0, RPT % C)],
                        out_sh.at[pl.ds(s * RPT + (RPT // C) * C, RPT % C)])
    plsc.subcore_barrier()

    def _stage_piece(base):
        pltpu.sync_copy(src_hbm.at[pl.ds(base, EB)], src_v)
        pltpu.sync_copy(dst_hbm.at[pl.ds(base, EB)], dst_v)
        pltpu.async_copy(es_hbm.at[src_v], esg_v, sem_g)
        pltpu.async_copy(ed_hbm.at[dst_v], edg_v, sem_g)
        pltpu.make_async_copy(es_hbm.at[pl.ds(0, EB)], esg_v, sem_g).wait()
        pltpu.make_async_copy(ed_hbm.at[pl.ds(0, EB)], edg_v, sem_g).wait()

    def _ex_vec(i):
        t = esg_v[pl.ds(i * L, L)] + edg_v[pl.ds(i * L, L)]
        lg = jnp.where(t > 0.0, t, t * NEG_SLOPE)
        return jnp.exp(lg)

    def _gather(j, buf):
        pltpu.async_copy(h_hbm.at[src_v.at[pl.ds(j * C, C)]], buf, sem_g)

    def _drain(buf, sem):
        pltpu.make_async_copy(h_hbm.at[pl.ds(0, C)], buf, sem).wait()

    def _scatter(j, buf, sem):
        pltpu.async_copy(buf, out_sh.at[dst_v.at[pl.ds(j * C, C)]],
                         sem, add=True)

    def _compute(j, buf):
        for k in range(C // L):
            idst = dst_v[pl.ds(j * C + k * L, L)]
            ex = _ex_vec(j * (C // L) + k)
            plsc.addupdate_scatter(den_v, _den_idx(idst), ex)
            ex_c[pl.ds(k * L, L)] = ex

        def _scale(r, carry2):
            a = plsc.load_gather(ex_c, [jnp.full((L,), r, jnp.int32)])
            for k in range(D // L):
                buf[r, pl.ds(k * L, L)] = buf[r, pl.ds(k * L, L)] * a
            return carry2
        lax.fori_loop(0, C, _scale, 0)

    for t in range(EPW // EB):
        _stage_piece(wid * EPW + t * EB)

        _gather(0, rows_a)

        def _trip(i, carry):
            j0 = 3 * i

            @pl.when(j0 >= 2)
            def _():
                _drain(rows_b, sem_s1)

            _drain(rows_a, sem_g)
            _gather(j0 + 1, rows_b)
            _compute(j0, rows_a)
            _scatter(j0, rows_a, sem_s0)

            @pl.when(j0 + 1 >= 2)
            def _():
                _drain(rows_c, sem_s2)

            _drain(rows_b, sem_g)
            _gather(j0 + 2, rows_c)
            _compute(j0 + 1, rows_b)
            _scatter(j0 + 1, rows_b, sem_s1)

            _drain(rows_a, sem_s0)
            _drain(rows_c, sem_g)
            _gather(j0 + 3, rows_a)
            _compute(j0 + 2, rows_c)
            _scatter(j0 + 2, rows_c, sem_s2)
            return carry
        lax.fori_loop(0, (NCH - 1) // 3, _trip, 0)
        _drain(rows_b, sem_s1)
        _drain(rows_a, sem_g)
        _compute(NCH - 1, rows_a)
        _scatter(NCH - 1, rows_a, sem_s0)
        _drain(rows_c, sem_s2)
        _drain(rows_a, sem_s0)

    for k in range(DEN_R // L):
        rows16 = lax.iota(jnp.int32, L) + k * L
        pltpu.sync_copy(den_v.at[pl.ds(k * L, L)],
                        den_sh.at[rows16], add=True)
    plsc.subcore_barrier()
    pltpu.sync_copy(out_sh.at[pl.ds(s * RPT, RPT)],
                    out_hbm.at[c, pl.ds(s * RPT, RPT)])

    @pl.when(s < DEN_R // 8)
    def _():
        pltpu.sync_copy(den_sh.at[pl.ds(s * 8, 8)],
                        dout_hbm.at[c, pl.ds(s * 8, 8)])


_sc_edge = pl.kernel(
    _sc_edge_body,
    out_type=(
        jax.ShapeDtypeStruct((NC, NP, D), jnp.float32),
        jax.ShapeDtypeStruct((NC, DEN_R, DEN_C), jnp.float32),
    ),
    mesh=plsc.VectorSubcoreMesh(core_axis_name="c", subcore_axis_name="s"),
    scratch_types=[
        pltpu.VMEM((DEN_R, DEN_C), jnp.float32),
        pltpu.VMEM((EB,), jnp.int32),
        pltpu.VMEM((EB,), jnp.int32),
        pltpu.VMEM((EB,), jnp.float32),
        pltpu.VMEM((EB,), jnp.float32),
        pltpu.VMEM((C,), jnp.float32),
        pltpu.VMEM((C, D), jnp.float32),
        pltpu.VMEM((C, D), jnp.float32),
        pltpu.VMEM((C, D), jnp.float32),
        pltpu.VMEM_SHARED((DEN_R, DEN_C), jnp.float32),
        pltpu.VMEM_SHARED((NP, D), jnp.float32),
        pltpu.SemaphoreType.DMA,
        pltpu.SemaphoreType.DMA,
        pltpu.SemaphoreType.DMA,
        pltpu.SemaphoreType.DMA,
    ],
    compiler_params=pltpu.CompilerParams(needs_layout_passes=False),
)


def _sc_layer(h, es, ed, src, dst):
    p, d = _sc_edge(h, es.reshape(N), ed.reshape(N), src, dst)
    return p[:, :N, :], d.reshape(NC, DEN_R * DEN_C, 1)[:, :N, :]


def kernel(x, edge_index, W1, a_src1, a_dst1, b1, W2, a_src2, a_dst2, b2,
           W3, a_src3, a_dst3, b3):
    src = edge_index[0]
    dst = edge_index[1]

    h, es, ed = _proj_first(x, W1, a_src1, a_dst1)
    p, d = _sc_layer(h, es, ed, src, dst)

    h, es, ed = _proj_next(p, d, b1, W2, a_src2, a_dst2)
    p, d = _sc_layer(h, es, ed, src, dst)

    h, es, ed = _proj_next(p, d, b2, W3, a_src3, a_dst3)
    p, d = _sc_layer(h, es, ed, src, dst)

    return _final(p, d, b3)


# speedup vs baseline: 42.8392x; 1.1790x over previous
import functools

import jax
import jax.numpy as jnp
from jax import lax
from jax.experimental import pallas as pl
from jax.experimental.pallas import tpu as pltpu
from jax.experimental.pallas import tpu_sc as plsc

N = 10000
E = 320000
D = 128
L = 16
NC = 2
NS = 16
NW = NC * NS
RPT = 632
LAST = N - (NS - 1) * RPT
EPW = E // NW
C = 80
EB = 2000
NCH = EB // C
DEN_R = 80
DEN_C = 128
NEG_SLOPE = 0.2
EPS = 1e-16


def _proj_first_body(x_ref, w_ref, as_ref, ad_ref, h_ref, es_ref, ed_ref):
    h = jnp.dot(x_ref[...], w_ref[...], preferred_element_type=jnp.float32)
    h_ref[...] = h
    es_ref[...] = jnp.dot(h, as_ref[...], preferred_element_type=jnp.float32)
    ed_ref[...] = jnp.dot(h, ad_ref[...], preferred_element_type=jnp.float32)


def _proj_next_body(p_ref, d_ref, b_ref, w_ref, as_ref, ad_ref,
                    h_ref, es_ref, ed_ref):
    den = d_ref[0] + d_ref[1] + EPS
    z = jnp.maximum((p_ref[0] + p_ref[1]) / den + b_ref[...], 0.0)
    h = jnp.dot(z, w_ref[...], preferred_element_type=jnp.float32)
    h_ref[...] = h
    es_ref[...] = jnp.dot(h, as_ref[...], preferred_element_type=jnp.float32)
    ed_ref[...] = jnp.dot(h, ad_ref[...], preferred_element_type=jnp.float32)


def _final_body(p_ref, d_ref, b_ref, o_ref):
    den = d_ref[0] + d_ref[1] + EPS
    o_ref[...] = (p_ref[0] + p_ref[1]) / den + b_ref[...]


_PROJ_OUT = (
    jax.ShapeDtypeStruct((N, D), jnp.float32),
    jax.ShapeDtypeStruct((N, 1), jnp.float32),
    jax.ShapeDtypeStruct((N, 1), jnp.float32),
)


def _proj_first(x, W, a_s, a_d):
    return pl.pallas_call(_proj_first_body, out_shape=_PROJ_OUT)(
        x, W, a_s.reshape(D, 1), a_d.reshape(D, 1))


def _proj_next(p, d, b, W, a_s, a_d):
    return pl.pallas_call(_proj_next_body, out_shape=_PROJ_OUT)(
        p, d, b.reshape(1, D), W, a_s.reshape(D, 1), a_d.reshape(D, 1))


def _final(p, d, b):
    return pl.pallas_call(
        _final_body, out_shape=jax.ShapeDtypeStruct((N, D), jnp.float32))(
            p, d, b.reshape(1, D))


def _den_idx(idst):
    return [lax.shift_right_logical(idst, 7), lax.bitwise_and(idst, 127)]


def _sc_edge_body(h_hbm, es_hbm, ed_hbm, src_hbm, dst_hbm,
                  out_hbm, dout_hbm,
                  den_v, src_v, dst_v, esg_v, edg_v,
                  rows_a, rows_b, rows_c, den_sh, out_sh, es_sh, ed_sh,
                  sem_g, sem_s0, sem_s1, sem_s2):
    c = lax.axis_index("c")
    s = lax.axis_index("s")
    wid = s * NC + c
    zero16 = jnp.zeros((L,), jnp.float32)

    def _zden(r, carry):
        for k in range(DEN_C // L):
            den_v[r, pl.ds(k * L, L)] = zero16
        return carry
    lax.fori_loop(0, DEN_R, _zden, 0)

    def _zrow(r, carry):
        for k in range(D // L):
            rows_a[r, pl.ds(k * L, L)] = zero16
        return carry
    lax.fori_loop(0, C, _zrow, 0)

    pltpu.sync_copy(den_v.at[pl.ds(s * (DEN_R // NS), DEN_R // NS)],
                    den_sh.at[pl.ds(s * (DEN_R // NS), DEN_R // NS)])

    @pl.when(s < 10)
    def _():
        pltpu.sync_copy(es_hbm.at[pl.ds(s * 1000, 1000)],
                        esg_v.at[pl.ds(0, 1000)])
        pltpu.sync_copy(ed_hbm.at[pl.ds(s * 1000, 1000)],
                        edg_v.at[pl.ds(0, 1000)])
        pltpu.sync_copy(esg_v.at[pl.ds(0, 1000)],
                        es_sh.at[pl.ds(s * 1000, 1000)])
        pltpu.sync_copy(edg_v.at[pl.ds(0, 1000)],
                        ed_sh.at[pl.ds(s * 1000, 1000)])

    for q in range(6):
        pltpu.sync_copy(rows_a, out_sh.at[pl.ds(s * RPT + q * C, C)])

    @pl.when(s < NS - 1)
    def _():
        pltpu.sync_copy(rows_a, out_sh.at[pl.ds(s * RPT + 6 * C, C)])
        pltpu.sync_copy(rows_a.at[pl.ds(0, RPT - 7 * C)],
                        out_sh.at[pl.ds(s * RPT + 7 * C, RPT - 7 * C)])

    @pl.when(s == NS - 1)
    def _():
        pltpu.sync_copy(rows_a.at[pl.ds(0, LAST - 6 * C)],
                        out_sh.at[pl.ds(s * RPT + 6 * C, LAST - 6 * C)])
    plsc.subcore_barrier()

    def _stage_piece(base):
        pltpu.sync_copy(src_hbm.at[pl.ds(base, EB)], src_v)
        pltpu.sync_copy(dst_hbm.at[pl.ds(base, EB)], dst_v)
        pltpu.async_copy(es_sh.at[src_v], esg_v, sem_g)
        pltpu.async_copy(ed_sh.at[dst_v], edg_v, sem_g)
        pltpu.make_async_copy(es_sh.at[pl.ds(0, EB)], esg_v, sem_g).wait()
        pltpu.make_async_copy(ed_sh.at[pl.ds(0, EB)], edg_v, sem_g).wait()

    def _ex_vec(i):
        t = esg_v[pl.ds(i * L, L)] + edg_v[pl.ds(i * L, L)]
        lg = jnp.where(t > 0.0, t, t * NEG_SLOPE)
        return jnp.exp(lg)

    def _gather(j, buf):
        pltpu.async_copy(h_hbm.at[src_v.at[pl.ds(j * C, C)]], buf, sem_g)

    def _drain(buf, sem):
        pltpu.make_async_copy(h_hbm.at[pl.ds(0, C)], buf, sem).wait()

    def _scatter(j, buf, sem):
        pltpu.async_copy(buf, out_sh.at[dst_v.at[pl.ds(j * C, C)]],
                         sem, add=True)

    def _compute(j, buf):
        for k in range(C // L):
            idst = dst_v[pl.ds(j * C + k * L, L)]
            ex = _ex_vec(j * (C // L) + k)
            plsc.addupdate_scatter(den_v, _den_idx(idst), ex)
            esg_v[pl.ds(j * C + k * L, L)] = ex

        def _scale(r, carry2):
            a = plsc.load_gather(esg_v, [jnp.full((L,), j * C + r, jnp.int32)])
            for k in range(D // L):
                buf[r, pl.ds(k * L, L)] = buf[r, pl.ds(k * L, L)] * a
            return carry2
        lax.fori_loop(0, C, _scale, 0)

    for t in range(EPW // EB):
        _stage_piece(wid * EPW + t * EB)

        _gather(0, rows_a)

        def _trip(i, carry):
            j0 = 3 * i

            @pl.when(j0 >= 2)
            def _():
                _drain(rows_b, sem_s1)

            _drain(rows_a, sem_g)
            _gather(j0 + 1, rows_b)
            _compute(j0, rows_a)
            _scatter(j0, rows_a, sem_s0)

            @pl.when(j0 + 1 >= 2)
            def _():
                _drain(rows_c, sem_s2)

            _drain(rows_b, sem_g)
            _gather(j0 + 2, rows_c)
            _compute(j0 + 1, rows_b)
            _scatter(j0 + 1, rows_b, sem_s1)

            _drain(rows_a, sem_s0)
            _drain(rows_c, sem_g)
            _gather(j0 + 3, rows_a)
            _compute(j0 + 2, rows_c)
            _scatter(j0 + 2, rows_c, sem_s2)
            return carry
        lax.fori_loop(0, (NCH - 1) // 3, _trip, 0)
        _drain(rows_b, sem_s1)
        _drain(rows_a, sem_g)
        _compute(NCH - 1, rows_a)
        _scatter(NCH - 1, rows_a, sem_s0)
        _drain(rows_c, sem_s2)
        _drain(rows_a, sem_s0)

    for k in range(DEN_R // L):
        rows16 = lax.iota(jnp.int32, L) + k * L
        pltpu.sync_copy(den_v.at[pl.ds(k * L, L)],
                        den_sh.at[rows16], add=True)
    plsc.subcore_barrier()
    @pl.when(s < NS - 1)
    def _():
        pltpu.sync_copy(out_sh.at[pl.ds(s * RPT, RPT)],
                        out_hbm.at[c, pl.ds(s * RPT, RPT)])

    @pl.when(s == NS - 1)
    def _():
        pltpu.sync_copy(out_sh.at[pl.ds(s * RPT, LAST)],
                        out_hbm.at[c, pl.ds(s * RPT, LAST)])

    @pl.when(s < DEN_R // 8)
    def _():
        pltpu.sync_copy(den_sh.at[pl.ds(s * 8, 8)],
                        dout_hbm.at[c, pl.ds(s * 8, 8)])


_sc_edge = pl.kernel(
    _sc_edge_body,
    out_type=(
        jax.ShapeDtypeStruct((NC, N, D), jnp.float32),
        jax.ShapeDtypeStruct((NC, DEN_R, DEN_C), jnp.float32),
    ),
    mesh=plsc.VectorSubcoreMesh(core_axis_name="c", subcore_axis_name="s"),
    scratch_types=[
        pltpu.VMEM((DEN_R, DEN_C), jnp.float32),
        pltpu.VMEM((EB,), jnp.int32),
        pltpu.VMEM((EB,), jnp.int32),
        pltpu.VMEM((EB,), jnp.float32),
        pltpu.VMEM((EB,), jnp.float32),
        pltpu.VMEM((C, D), jnp.float32),
        pltpu.VMEM((C, D), jnp.float32),
        pltpu.VMEM((C, D), jnp.float32),
        pltpu.VMEM_SHARED((DEN_R, DEN_C), jnp.float32),
        pltpu.VMEM_SHARED((N, D), jnp.float32),
        pltpu.VMEM_SHARED((N,), jnp.float32),
        pltpu.VMEM_SHARED((N,), jnp.float32),
        pltpu.SemaphoreType.DMA,
        pltpu.SemaphoreType.DMA,
        pltpu.SemaphoreType.DMA,
        pltpu.SemaphoreType.DMA,
    ],
    compiler_params=pltpu.CompilerParams(needs_layout_passes=False),
)


def _sc_layer(h, es, ed, src, dst):
    p, d = _sc_edge(h, es.reshape(N), ed.reshape(N), src, dst)
    return p, d.reshape(NC, DEN_R * DEN_C, 1)[:, :N, :]


def kernel(x, edge_index, W1, a_src1, a_dst1, b1, W2, a_src2, a_dst2, b2,
           W3, a_src3, a_dst3, b3):
    src = edge_index[0]
    dst = edge_index[1]

    h, es, ed = _proj_first(x, W1, a_src1, a_dst1)
    p, d = _sc_layer(h, es, ed, src, dst)

    h, es, ed = _proj_next(p, d, b1, W2, a_src2, a_dst2)
    p, d = _sc_layer(h, es, ed, src, dst)

    h, es, ed = _proj_next(p, d, b2, W3, a_src3, a_dst3)
    p, d = _sc_layer(h, es, ed, src, dst)

    return _final(p, d, b3)
